# speedup vs baseline: 1.6637x; 1.6637x over previous
import functools

import jax
import jax.numpy as jnp
from jax import lax
from jax.experimental import pallas as pl
from jax.experimental.pallas import tpu as pltpu
from jax.experimental.pallas import tpu_sc as plsc

N = 10000
K = 32
D = 128
NW = 32
N_PAD = 10240
R = N_PAD // NW
CHUNK_NODES = 4
CHUNK_ROWS = CHUNK_NODES * K
N_CHUNKS = R // CHUNK_NODES
LANES = 16

_mesh = plsc.VectorSubcoreMesh(core_axis_name="c", subcore_axis_name="s")


def _splat(vec, j):
    idx = jnp.full((LANES, 1), j, dtype=jnp.int32)
    return lax.gather(
        vec, idx,
        dimension_numbers=lax.GatherDimensionNumbers(
            offset_dims=(), collapsed_slice_dims=(0,), start_index_map=(0,)),
        slice_sizes=(1,),
        mode=lax.GatherScatterMode.PROMISE_IN_BOUNDS)


@functools.partial(
    pl.kernel,
    mesh=_mesh,
    out_type=jax.ShapeDtypeStruct((N_PAD, D), jnp.float32),
    scratch_types=[
        pltpu.VMEM((N_CHUNKS, CHUNK_ROWS), jnp.int32),
        pltpu.VMEM((R * K,), jnp.float32),
        pltpu.VMEM((CHUNK_ROWS, D), jnp.float32),
        pltpu.VMEM((R, D), jnp.float32),
        pltpu.SemaphoreType.DMA,
    ],
)
def _sc_agg(h_hbm, idx_hbm, w_hbm, agg_hbm, idx_v, w_v, rows, out_v, sem):
    wid = lax.axis_index("s") * 2 + lax.axis_index("c")
    pltpu.sync_copy(idx_hbm.at[pl.ds(wid * N_CHUNKS, N_CHUNKS)], idx_v)
    pltpu.sync_copy(w_hbm.at[pl.ds(wid * (R * K), R * K)], w_v)

    def chunk_body(g, carry):
        pltpu.async_copy(h_hbm.at[idx_v.at[g]], rows, sem).wait()

        def node_body(nl, c2):
            n = g * CHUNK_NODES + nl
            woff = n * K
            wv0 = w_v[pl.ds(woff, LANES)]
            wv1 = w_v[pl.ds(woff + LANES, LANES)]
            acc = [jnp.zeros((LANES,), jnp.float32) for _ in range(8)]
            for k in range(K):
                s = _splat(wv0 if k < LANES else wv1, k % LANES)
                r = nl * K + k
                for dd in range(8):
                    acc[dd] = acc[dd] + s * rows[r, pl.ds(dd * LANES, LANES)]
            for dd in range(8):
                out_v[n, pl.ds(dd * LANES, LANES)] = acc[dd]
            return c2

        return lax.fori_loop(0, CHUNK_NODES, node_body, carry)

    lax.fori_loop(0, N_CHUNKS, chunk_body, 0)
    pltpu.sync_copy(out_v, agg_hbm.at[pl.ds(wid * R, R)])


BLK = 512
GRID = N_PAD // BLK


def _tc_body(h_ref, a_ref, ws_ref, wn_ref, g_ref, b_ref, o_ref):
    x = h_ref[...]
    a = a_ref[...]
    y = jnp.dot(x, ws_ref[...], preferred_element_type=jnp.float32)
    y = y + jnp.dot(a, wn_ref[...], preferred_element_type=jnp.float32)
    y = 0.5 * y * (1.0 + lax.erf(y * 0.7071067811865476))
    mu = jnp.mean(y, axis=-1, keepdims=True)
    var = jnp.mean((y - mu) ** 2, axis=-1, keepdims=True)
    o_ref[...] = (y - mu) * lax.rsqrt(var + 1e-5) * g_ref[...] + b_ref[...]


def _tc_call(h, agg, ws_t, wn_t, gamma, beta):
    return pl.pallas_call(
        _tc_body,
        grid=(GRID,),
        in_specs=[
            pl.BlockSpec((BLK, D), lambda i: (i, 0)),
            pl.BlockSpec((BLK, D), lambda i: (i, 0)),
            pl.BlockSpec((D, D), lambda i: (0, 0)),
            pl.BlockSpec((D, D), lambda i: (0, 0)),
            pl.BlockSpec((1, D), lambda i: (0, 0)),
            pl.BlockSpec((1, D), lambda i: (0, 0)),
        ],
        out_specs=pl.BlockSpec((BLK, D), lambda i: (i, 0)),
        out_shape=jax.ShapeDtypeStruct((N, D), jnp.float32),
    )(h, agg, ws_t, wn_t, gamma, beta)


def kernel(h, nbr_idx, nbr_w, W_self, W_nei, gamma, beta):
    pad = N_PAD - N
    idx_pad = jnp.pad(nbr_idx.astype(jnp.int32), ((0, pad), (0, 0)))
    idx_pad = idx_pad.reshape(NW * N_CHUNKS, CHUNK_ROWS)
    w_pad = jnp.pad(nbr_w, ((0, pad), (0, 0))).reshape(-1)
    agg = _sc_agg(h, idx_pad, w_pad)
    return _tc_call(h, agg, W_self.T, W_nei.T,
                    gamma.reshape(1, D), beta.reshape(1, D))
